# Initial kernel scaffold; baseline (speedup 1.0000x reference)
import jax
import jax.numpy as jnp
from jax.experimental import pallas as pl


def kernel(flat, cu_seqlens):
    raise NotImplementedError("write your pallas kernel here")


# speedup vs baseline: 4.3374x; 4.3374x over previous
import functools

import jax
import jax.numpy as jnp
from jax import lax
from jax.experimental import pallas as pl
from jax.experimental.pallas import tpu as pltpu
from jax.experimental.pallas import tpu_sc as plsc

TARGET_SAMPLES = 4096
LANES = 16

_info = plsc.get_sparse_core_info()
NC = _info.num_cores
NS = _info.num_subcores
NW = NC * NS


def _make_padder(total, batch, pad_total, cu_pad):
    out_len = batch * TARGET_SAMPLES
    ch = out_len // NW
    wpr = TARGET_SAMPLES // ch
    chunk_len = ch + LANES

    mesh = plsc.VectorSubcoreMesh(core_axis_name="c", subcore_axis_name="s")

    @functools.partial(
        pl.kernel,
        mesh=mesh,
        out_type=jax.ShapeDtypeStruct((out_len,), jnp.float32),
        scratch_types=[
            pltpu.VMEM((cu_pad,), jnp.int32),
            pltpu.VMEM((chunk_len,), jnp.float32),
            pltpu.VMEM((ch,), jnp.float32),
        ],
    )
    def padder(flat_hbm, cu_hbm, out_hbm, cu_v, chunk_v, out_v):
        w = lax.axis_index("s") * NC + lax.axis_index("c")
        b = w // wpr
        i_start = (w % wpr) * ch

        pltpu.sync_copy(cu_hbm, cu_v)
        lanes = lax.iota(jnp.int32, LANES)
        cu_win = cu_v[pl.ds(b, LANES)]
        cu_b = cu_win[0]
        cu_b1 = cu_win[1]
        rel_len = cu_b1 - cu_b - i_start

        src = cu_b + i_start
        s0 = (src // 8) * 8
        r = src - s0
        pltpu.sync_copy(flat_hbm.at[pl.ds(s0, chunk_len)], chunk_v)

        for j in range(ch // LANES):
            vals = chunk_v[pl.ds(r + j * LANES, LANES)]
            ok = (lanes + (j * LANES)) < rel_len
            out_v[pl.ds(j * LANES, LANES)] = jnp.where(ok, vals, 0.0)

        pltpu.sync_copy(out_v, out_hbm.at[pl.ds(w * ch, ch)])

    return padder


def kernel(flat, cu_seqlens):
    total = flat.shape[0]
    batch = cu_seqlens.shape[0] - 1
    flat1 = flat.reshape(total)
    pad_total = total + TARGET_SAMPLES + 2 * LANES
    flat_pad = jnp.pad(flat1, (0, pad_total - total))
    cu_pad = ((batch + 2 * LANES - 1) // LANES) * LANES
    cu_padded = jnp.pad(cu_seqlens, (0, cu_pad - cu_seqlens.shape[0]))
    out = _make_padder(total, batch, pad_total, cu_pad)(flat_pad, cu_padded)
    return out.reshape(batch, TARGET_SAMPLES, 1)
